# baseline (speedup 1.0000x reference)
import jax
import jax.numpy as jnp
from jax import lax
from jax.experimental import pallas as pl

STEPS_PER_DAY = 288
TE_DIM = 32
NUM_NODES = 170
SE_DIM = 16
OUT_DIM = SE_DIM + TE_DIM
ND = NUM_NODES * OUT_DIM


def _body(tod_ref, dow_ref, a_ref, tw_ref, dw_ref, o_ref):
    bsz = tod_ref.shape[-1]
    tod = jnp.clip(tod_ref[0], 0, STEPS_PER_DAY - 1)
    dow = jnp.clip(dow_ref[0], 0, 6)
    oh_t = (tod == lax.broadcasted_iota(jnp.int32, (STEPS_PER_DAY, bsz), 0))
    oh_d = (dow == lax.broadcasted_iota(jnp.int32, (8, bsz), 0))
    te_t = lax.dot(tw_ref[...], oh_t.astype(jnp.float32),
                   preferred_element_type=jnp.float32)
    te_t = te_t + lax.dot(dw_ref[...], oh_d.astype(jnp.float32),
                          preferred_element_type=jnp.float32)
    bmat = jnp.concatenate(
        [jnp.ones((1, bsz), jnp.float32),
         jnp.zeros((SE_DIM, bsz), jnp.float32),
         te_t], axis=0)
    o_ref[0] = lax.dot(a_ref[...], bmat, preferred_element_type=jnp.float32,
                       precision=lax.Precision.HIGHEST)


@jax.jit
def kernel(te, se, tod_weight, dow_weight):
    b, t = te.shape[0], te.shape[1]
    tod_ids = te[..., 0].astype(jnp.int32).T.reshape(t, 1, b)
    dow_ids = te[..., 1].astype(jnp.int32).T.reshape(t, 1, b)
    tod_wT = tod_weight.T
    dow_wT = jnp.pad(dow_weight, ((0, 1), (0, 0))).T
    se_pad = jnp.pad(se, ((0, 0), (0, TE_DIM)))
    amat = jnp.concatenate(
        [se_pad.reshape(ND, 1),
         jnp.tile(jnp.eye(OUT_DIM, dtype=jnp.float32), (NUM_NODES, 1))],
        axis=1)

    out = pl.pallas_call(
        _body,
        grid=(t,),
        in_specs=[
            pl.BlockSpec((1, 1, b), lambda i: (i, 0, 0)),
            pl.BlockSpec((1, 1, b), lambda i: (i, 0, 0)),
            pl.BlockSpec((ND, TE_DIM + SE_DIM + 1), lambda i: (0, 0)),
            pl.BlockSpec((TE_DIM, STEPS_PER_DAY), lambda i: (0, 0)),
            pl.BlockSpec((TE_DIM, 8), lambda i: (0, 0)),
        ],
        out_specs=pl.BlockSpec((1, ND, b), lambda i: (i, 0, 0)),
        out_shape=jax.ShapeDtypeStruct((t, ND, b), jnp.float32),
    )(tod_ids, dow_ids, amat, tod_wT, dow_wT)
    out = out.reshape(t, NUM_NODES, OUT_DIM, b)
    return jnp.transpose(out, (3, 0, 1, 2))


# speedup vs baseline: 2.1704x; 2.1704x over previous
import jax
import jax.numpy as jnp
from jax import lax
from jax.experimental import pallas as pl

STEPS_PER_DAY = 288
TE_DIM = 32
NUM_NODES = 170
SE_DIM = 16
OUT_DIM = SE_DIM + TE_DIM
ND = NUM_NODES * OUT_DIM


def _body(tod_ref, dow_ref, se_ref, a_ref, tw_ref, dw_ref, o_ref):
    bsz = tod_ref.shape[-1]
    tod = jnp.clip(tod_ref[0], 0, STEPS_PER_DAY - 1)
    dow = jnp.clip(dow_ref[0], 0, 6)
    oh_t = (tod == lax.broadcasted_iota(jnp.int32, (STEPS_PER_DAY, bsz), 0))
    oh_d = (dow == lax.broadcasted_iota(jnp.int32, (8, bsz), 0))
    te_t = lax.dot(tw_ref[...], oh_t.astype(jnp.float32),
                   preferred_element_type=jnp.float32)
    te_t = te_t + lax.dot(dw_ref[...], oh_d.astype(jnp.float32),
                          preferred_element_type=jnp.float32)
    bmat = jnp.concatenate(
        [jnp.zeros((SE_DIM, bsz), jnp.float32), te_t], axis=0)
    o_ref[0] = lax.dot(a_ref[...], bmat,
                       preferred_element_type=jnp.float32) + se_ref[...]


@jax.jit
def kernel(te, se, tod_weight, dow_weight):
    b, t = te.shape[0], te.shape[1]
    tod_ids = te[..., 0].astype(jnp.int32).T.reshape(t, 1, b)
    dow_ids = te[..., 1].astype(jnp.int32).T.reshape(t, 1, b)
    tod_wT = tod_weight.T
    dow_wT = jnp.pad(dow_weight, ((0, 1), (0, 0))).T
    se_col = jnp.pad(se, ((0, 0), (0, TE_DIM))).reshape(ND, 1)
    amat = jnp.tile(jnp.eye(OUT_DIM, dtype=jnp.float32),
                    (NUM_NODES, 1))

    out = pl.pallas_call(
        _body,
        grid=(t,),
        in_specs=[
            pl.BlockSpec((1, 1, b), lambda i: (i, 0, 0)),
            pl.BlockSpec((1, 1, b), lambda i: (i, 0, 0)),
            pl.BlockSpec((ND, 1), lambda i: (0, 0)),
            pl.BlockSpec((ND, OUT_DIM), lambda i: (0, 0)),
            pl.BlockSpec((TE_DIM, STEPS_PER_DAY), lambda i: (0, 0)),
            pl.BlockSpec((TE_DIM, 8), lambda i: (0, 0)),
        ],
        out_specs=pl.BlockSpec((1, ND, b), lambda i: (i, 0, 0)),
        out_shape=jax.ShapeDtypeStruct((t, ND, b), jnp.float32),
    )(tod_ids, dow_ids, se_col, amat, tod_wT, dow_wT)
    out = out.reshape(t, NUM_NODES, OUT_DIM, b)
    return jnp.transpose(out, (3, 0, 1, 2))
